# baseline (speedup 1.0000x reference)
import functools

import jax
import jax.numpy as jnp
from jax import lax
from jax.experimental import pallas as pl
from jax.experimental.pallas import tpu as pltpu
from jax.experimental.pallas import tpu_sc as plsc

NC = 2
NS = 16
NW = NC * NS
K = 80
L = 16
CP = 80
NB = 3
ZB = 20


def _make_sc_edge(n_nodes: int, n_edges: int, c_feat: int):
    assert n_edges % (NW * K) == 0
    nrows = n_edges // K
    r2 = nrows // NW
    zrow = 640
    n_full, n_rem = divmod(n_nodes, zrow)
    grp = K // L
    mesh = plsc.VectorSubcoreMesh(core_axis_name="c", subcore_axis_name="s",
                                  num_cores=NC, num_subcores=NS)

    @functools.partial(
        pl.kernel,
        out_type=jax.ShapeDtypeStruct((NC, n_nodes, CP), jnp.float32),
        mesh=mesh,
        scratch_types=dict(
            asrc_l=pltpu.VMEM((n_nodes,), jnp.float32),
            adst_l=pltpu.VMEM((n_nodes,), jnp.float32),
            src2_l=pltpu.VMEM((r2, K), jnp.int32),
            dst2_l=pltpu.VMEM((r2, K), jnp.int32),
            gbuf=pltpu.VMEM((NB, K, c_feat), jnp.float32),
            sbuf=pltpu.VMEM((NB, K, CP), jnp.float32),
            zbuf=pltpu.VMEM((ZB, CP), jnp.float32),
            out_sp=pltpu.VMEM_SHARED((n_nodes, CP), jnp.float32),
            gsem=pltpu.SemaphoreType.DMA((NB,)),
            ssem=pltpu.SemaphoreType.DMA((NB,)),
        ),
        compiler_params=pltpu.CompilerParams(needs_layout_passes=False,
                                            use_tc_tiling_on_sc=False),
    )
    def sc_edge(src2_hbm, dst2_hbm, h_hbm, asrc_hbm, adst_hbm, out_hbm,
                asrc_l, adst_l, src2_l, dst2_l, gbuf, sbuf, zbuf, out_sp,
                gsem, ssem):
        c = lax.axis_index("c")
        s = lax.axis_index("s")
        w = c * NS + s

        pltpu.sync_copy(asrc_hbm, asrc_l)
        pltpu.sync_copy(adst_hbm, adst_l)
        pltpu.sync_copy(src2_hbm.at[w], src2_l)
        pltpu.sync_copy(dst2_hbm.at[w], dst2_l)

        zero16 = jnp.zeros((L,), jnp.float32)
        for i in range(ZB):
            for rr in range(CP // L):
                zbuf[i, pl.ds(rr * L, L)] = zero16

        @pl.when(s < n_full)
        def _():
            def zc(i, cr):
                pltpu.sync_copy(zbuf, out_sp.at[pl.ds(s * zrow + i * ZB, ZB)])
                return cr
            lax.fori_loop(0, zrow // ZB, zc, 0)

        @pl.when(s == n_full)
        def _():
            if n_rem:
                def zc(i, cr):
                    pltpu.sync_copy(
                        zbuf, out_sp.at[pl.ds(n_full * zrow + i * ZB, ZB)])
                    return cr
                lax.fori_loop(0, n_rem // ZB, zc, 0)

        plsc.subcore_barrier()

        onehot0 = (lax.iota(jnp.int32, L) == 0).astype(jnp.float32)

        def scale_chunk(j, src, dst):
            for g in range(grp):
                sidx = src2_l[j, pl.ds(g * L, L)]
                didx = dst2_l[j, pl.ds(g * L, L)]
                e = (plsc.load_gather(asrc_l, [sidx])
                     + plsc.load_gather(adst_l, [didx]))
                e = jnp.maximum(e, 0.2 * e)
                exv = jnp.exp(e)
                for ii in range(L):
                    a = exv[ii]
                    i = g * L + ii
                    for rr in range(c_feat // L):
                        dst[i, pl.ds(rr * L, L)] = src[i, pl.ds(rr * L, L)] * a
                    dst[i, pl.ds(c_feat, L)] = onehot0 * a

        def issue_gather(j, b):
            return pltpu.async_copy(h_hbm.at[src2_l.at[j]], gbuf.at[b],
                                    gsem.at[b])

        def issue_scatter(j, b):
            return pltpu.async_copy(sbuf.at[b], out_sp.at[dst2_l.at[j]],
                                    ssem.at[b], add=True)

        def wait_gather(j, b):
            pltpu.make_async_copy(h_hbm.at[src2_l.at[j]], gbuf.at[b],
                                  gsem.at[b]).wait()

        def wait_scatter(j, b):
            pltpu.make_async_copy(sbuf.at[b], out_sp.at[dst2_l.at[j]],
                                  ssem.at[b]).wait()

        def prologue(b, carry):
            issue_gather(b, b)
            return carry

        lax.fori_loop(0, NB, prologue, 0)

        def main_body(j, carry):
            b = lax.rem(j, NB)
            pl.when(j >= NB)(lambda: wait_scatter(j, b))
            wait_gather(j, b)
            scale_chunk(j, gbuf.at[b], sbuf.at[b])
            def _prefetch():
                issue_gather(j + NB, b)
            pl.when(j + NB < r2)(_prefetch)
            issue_scatter(j, b)
            return carry

        lax.fori_loop(0, r2, main_body, 0)

        def epilogue(t, carry):
            j = r2 - NB + t
            wait_scatter(j, lax.rem(j, NB))
            return carry

        lax.fori_loop(0, NB, epilogue, 0)
        plsc.subcore_barrier()

        @pl.when(s < n_full)
        def _():
            pltpu.sync_copy(out_sp.at[pl.ds(s * zrow, zrow)],
                            out_hbm.at[c, pl.ds(s * zrow, zrow)])

        @pl.when(s == n_full)
        def _():
            if n_rem:
                pltpu.sync_copy(out_sp.at[pl.ds(n_full * zrow, n_rem)],
                                out_hbm.at[c, pl.ds(n_full * zrow, n_rem)])

    return sc_edge


_BLK = 2000


def _proj1_body(x_ref, w_ref, asv_ref, adv_ref, h_ref, s_ref, d_ref):
    h = jnp.dot(x_ref[...], w_ref[...], preferred_element_type=jnp.float32)
    h_ref[...] = h
    s_ref[...] = jnp.dot(h, asv_ref[...], preferred_element_type=jnp.float32)
    d_ref[...] = jnp.dot(h, adv_ref[...], preferred_element_type=jnp.float32)


def _tc_proj1(x, w, asv, adv):
    n, d = x.shape
    c = w.shape[1]
    grid = n // _BLK
    return pl.pallas_call(
        _proj1_body,
        grid=(grid,),
        in_specs=[
            pl.BlockSpec((_BLK, d), lambda i: (i, 0)),
            pl.BlockSpec((d, c), lambda i: (0, 0)),
            pl.BlockSpec((c, 1), lambda i: (0, 0)),
            pl.BlockSpec((c, 1), lambda i: (0, 0)),
        ],
        out_specs=[
            pl.BlockSpec((_BLK, c), lambda i: (i, 0)),
            pl.BlockSpec((_BLK, 1), lambda i: (i, 0)),
            pl.BlockSpec((_BLK, 1), lambda i: (i, 0)),
        ],
        out_shape=[
            jax.ShapeDtypeStruct((n, c), jnp.float32),
            jax.ShapeDtypeStruct((n, 1), jnp.float32),
            jax.ShapeDtypeStruct((n, 1), jnp.float32),
        ],
    )(x, w, asv, adv)


def _gat_out(p_ref, b_ref, c):
    num = p_ref[0, :, :c] + p_ref[1, :, :c]
    den = p_ref[0, :, c:c + 1] + p_ref[1, :, c:c + 1]
    out = num / (den + 1e-16)
    return jax.nn.relu(out + b_ref[...])


def _proj2_body(p_ref, b_ref, w_ref, asv_ref, adv_ref, h_ref, s_ref, d_ref):
    c = w_ref.shape[0]
    hin = _gat_out(p_ref, b_ref, c)
    h = jnp.dot(hin, w_ref[...], preferred_element_type=jnp.float32)
    h_ref[...] = h
    s_ref[...] = jnp.dot(h, asv_ref[...], preferred_element_type=jnp.float32)
    d_ref[...] = jnp.dot(h, adv_ref[...], preferred_element_type=jnp.float32)


def _tc_proj2(p, b, w, asv, adv):
    _, n, _ = p.shape
    c = w.shape[0]
    c2 = w.shape[1]
    grid = n // _BLK
    return pl.pallas_call(
        _proj2_body,
        grid=(grid,),
        in_specs=[
            pl.BlockSpec((2, _BLK, CP), lambda i: (0, i, 0)),
            pl.BlockSpec((1, c), lambda i: (0, 0)),
            pl.BlockSpec((c, c2), lambda i: (0, 0)),
            pl.BlockSpec((c2, 1), lambda i: (0, 0)),
            pl.BlockSpec((c2, 1), lambda i: (0, 0)),
        ],
        out_specs=[
            pl.BlockSpec((_BLK, c2), lambda i: (i, 0)),
            pl.BlockSpec((_BLK, 1), lambda i: (i, 0)),
            pl.BlockSpec((_BLK, 1), lambda i: (i, 0)),
        ],
        out_shape=[
            jax.ShapeDtypeStruct((n, c2), jnp.float32),
            jax.ShapeDtypeStruct((n, 1), jnp.float32),
            jax.ShapeDtypeStruct((n, 1), jnp.float32),
        ],
    )(p, b, w, asv, adv)


def _final_body(p_ref, b_ref, wp_ref, bp_ref, bat_ref, out_ref, cnt_ref, acc_ref):
    i = pl.program_id(0)
    nsteps = pl.num_programs(0)
    g = cnt_ref.shape[0]
    c = wp_ref.shape[0]

    hin = _gat_out(p_ref, b_ref, c)
    sc = jnp.dot(hin, wp_ref[...], preferred_element_type=jnp.float32)
    sc = sc + bp_ref[...]
    iota_g = lax.broadcasted_iota(jnp.int32, (_BLK, g), 1)
    oh = (bat_ref[...] == iota_g).astype(jnp.float32)
    part = lax.dot_general(oh, sc, (((0,), (0,)), ((), ())),
                           preferred_element_type=jnp.float32)
    cnt = jnp.sum(oh, axis=0)[:, None]

    @pl.when(i == 0)
    def _():
        cnt_ref[...] = jnp.zeros_like(cnt_ref)
        acc_ref[...] = jnp.zeros_like(acc_ref)

    cnt_ref[...] += cnt
    acc_ref[...] += part

    @pl.when(i == nsteps - 1)
    def _():
        out_ref[...] = acc_ref[...] / jnp.maximum(cnt_ref[...], 1.0)


def _tc_final(p, b, wp, bp, bat, g):
    _, n, _ = p.shape
    c = wp.shape[0]
    out_c = wp.shape[1]
    grid = n // _BLK
    return pl.pallas_call(
        _final_body,
        grid=(grid,),
        in_specs=[
            pl.BlockSpec((2, _BLK, CP), lambda i: (0, i, 0)),
            pl.BlockSpec((1, c), lambda i: (0, 0)),
            pl.BlockSpec((c, out_c), lambda i: (0, 0)),
            pl.BlockSpec((1, out_c), lambda i: (0, 0)),
            pl.BlockSpec((_BLK, 1), lambda i: (i, 0)),
        ],
        out_specs=pl.BlockSpec((g, out_c), lambda i: (0, 0)),
        out_shape=jax.ShapeDtypeStruct((g, out_c), jnp.float32),
        scratch_shapes=[
            pltpu.VMEM((g, 1), jnp.float32),
            pltpu.VMEM((g, out_c), jnp.float32),
        ],
    )(p, b, wp, bp, bat)


@jax.jit
def kernel(x, edge_index, batch, W1, a_src1, a_dst1, b1,
           W2, a_src2, a_dst2, b2, Wp, bp):
    n, _ = x.shape
    e = edge_index.shape[1]
    c = W1.shape[1]
    g = 64

    nrows = e // K
    src2 = edge_index[0].reshape(NW, nrows // NW, K)
    dst2 = edge_index[1].reshape(NW, nrows // NW, K)

    sc_edge = _make_sc_edge(n, e, c)

    h1, as1, ad1 = _tc_proj1(x, W1, a_src1.reshape(-1, 1), a_dst1.reshape(-1, 1))
    p1 = sc_edge(src2, dst2, h1, as1.reshape(-1), ad1.reshape(-1))
    h2, as2, ad2 = _tc_proj2(p1, b1.reshape(1, -1), W2,
                             a_src2.reshape(-1, 1), a_dst2.reshape(-1, 1))
    p2 = sc_edge(src2, dst2, h2, as2.reshape(-1), ad2.reshape(-1))
    out = _tc_final(p2, b2.reshape(1, -1), Wp, bp.reshape(1, -1),
                    batch.reshape(-1, 1), g)
    return out


# speedup vs baseline: 1.9418x; 1.9418x over previous
import functools

import jax
import jax.numpy as jnp
from jax import lax
from jax.experimental import pallas as pl
from jax.experimental.pallas import tpu as pltpu
from jax.experimental.pallas import tpu_sc as plsc

NC = 2
NS = 16
NW = NC * NS
K = 80
L = 16
CP = 80
NB = 3
ZB = 20


def _make_sc_edge(n_nodes: int, n_edges: int, c_feat: int):
    assert n_edges % (NW * K) == 0
    nrows = n_edges // K
    r2 = nrows // NW
    zrow = 640
    n_full, n_rem = divmod(n_nodes, zrow)
    grp = K // L
    mesh = plsc.VectorSubcoreMesh(core_axis_name="c", subcore_axis_name="s",
                                  num_cores=NC, num_subcores=NS)

    @functools.partial(
        pl.kernel,
        out_type=jax.ShapeDtypeStruct((NC, n_nodes, CP), jnp.float32),
        mesh=mesh,
        scratch_types=dict(
            asrc_l=pltpu.VMEM((n_nodes,), jnp.float32),
            adst_l=pltpu.VMEM((n_nodes,), jnp.float32),
            src2_l=pltpu.VMEM((r2, K), jnp.int32),
            dst2_l=pltpu.VMEM((r2, K), jnp.int32),
            gbuf=pltpu.VMEM((NB, K, CP), jnp.float32),
            sbuf=pltpu.VMEM((NB, K, CP), jnp.float32),
            zbuf=pltpu.VMEM((ZB, CP), jnp.float32),
            out_sp=pltpu.VMEM_SHARED((n_nodes, CP), jnp.float32),
            gsem=pltpu.SemaphoreType.DMA((NB,)),
            ssem=pltpu.SemaphoreType.DMA((NB,)),
        ),
        compiler_params=pltpu.CompilerParams(needs_layout_passes=False,
                                            use_tc_tiling_on_sc=False),
    )
    def sc_edge(src2_hbm, dst2_hbm, h_hbm, asrc_hbm, adst_hbm, out_hbm,
                asrc_l, adst_l, src2_l, dst2_l, gbuf, sbuf, zbuf, out_sp,
                gsem, ssem):
        c = lax.axis_index("c")
        s = lax.axis_index("s")
        w = c * NS + s

        pltpu.sync_copy(asrc_hbm, asrc_l)
        pltpu.sync_copy(adst_hbm, adst_l)
        pltpu.sync_copy(src2_hbm.at[w], src2_l)
        pltpu.sync_copy(dst2_hbm.at[w], dst2_l)

        zero16 = jnp.zeros((L,), jnp.float32)
        for i in range(ZB):
            for rr in range(CP // L):
                zbuf[i, pl.ds(rr * L, L)] = zero16

        @pl.when(s < n_full)
        def _():
            def zc(i, cr):
                pltpu.sync_copy(zbuf, out_sp.at[pl.ds(s * zrow + i * ZB, ZB)])
                return cr
            lax.fori_loop(0, zrow // ZB, zc, 0)

        @pl.when(s == n_full)
        def _():
            if n_rem:
                def zc(i, cr):
                    pltpu.sync_copy(
                        zbuf, out_sp.at[pl.ds(n_full * zrow + i * ZB, ZB)])
                    return cr
                lax.fori_loop(0, n_rem // ZB, zc, 0)

        plsc.subcore_barrier()

        onehot0 = (lax.iota(jnp.int32, L) == 0).astype(jnp.float32)

        def scale_chunk(j, src, dst):
            for g in range(grp):
                sidx = src2_l[j, pl.ds(g * L, L)]
                didx = dst2_l[j, pl.ds(g * L, L)]
                e = (plsc.load_gather(asrc_l, [sidx])
                     + plsc.load_gather(adst_l, [didx]))
                e = jnp.maximum(e, 0.2 * e)
                exv = jnp.exp(e)
                for ii in range(L):
                    a = exv[ii]
                    i = g * L + ii
                    for rr in range(c_feat // L):
                        dst[i, pl.ds(rr * L, L)] = src[i, pl.ds(rr * L, L)] * a
                    dst[i, pl.ds(c_feat, L)] = onehot0 * a

        def issue_gather(j, b):
            return pltpu.async_copy(h_hbm.at[src2_l.at[j]], gbuf.at[b],
                                    gsem.at[b])

        def issue_scatter(j, b):
            return pltpu.async_copy(sbuf.at[b], out_sp.at[dst2_l.at[j]],
                                    ssem.at[b], add=True)

        def wait_gather(j, b):
            pltpu.make_async_copy(h_hbm.at[src2_l.at[j]], gbuf.at[b],
                                  gsem.at[b]).wait()

        def wait_scatter(j, b):
            pltpu.make_async_copy(sbuf.at[b], out_sp.at[dst2_l.at[j]],
                                  ssem.at[b]).wait()

        def prologue(b, carry):
            issue_gather(b, b)
            return carry

        lax.fori_loop(0, NB, prologue, 0)

        def main_body(j, carry):
            b = lax.rem(j, NB)
            pl.when(j >= NB)(lambda: wait_scatter(j, b))
            wait_gather(j, b)
            scale_chunk(j, gbuf.at[b], sbuf.at[b])
            def _prefetch():
                issue_gather(j + NB, b)
            pl.when(j + NB < r2)(_prefetch)
            issue_scatter(j, b)
            return carry

        lax.fori_loop(0, r2, main_body, 0)

        def epilogue(t, carry):
            j = r2 - NB + t
            wait_scatter(j, lax.rem(j, NB))
            return carry

        lax.fori_loop(0, NB, epilogue, 0)
        plsc.subcore_barrier()

        @pl.when(s < n_full)
        def _():
            pltpu.sync_copy(out_sp.at[pl.ds(s * zrow, zrow)],
                            out_hbm.at[c, pl.ds(s * zrow, zrow)])

        @pl.when(s == n_full)
        def _():
            if n_rem:
                pltpu.sync_copy(out_sp.at[pl.ds(n_full * zrow, n_rem)],
                                out_hbm.at[c, pl.ds(n_full * zrow, n_rem)])

    return sc_edge


_BLK = 2000


def _proj1_body(x_ref, w_ref, asv_ref, adv_ref, h_ref, s_ref, d_ref):
    h = jnp.dot(x_ref[...], w_ref[...], preferred_element_type=jnp.float32)
    h_ref[...] = jnp.concatenate(
        [h, jnp.zeros((h.shape[0], CP - h.shape[1]), jnp.float32)], axis=1)
    s_ref[...] = jnp.dot(h, asv_ref[...], preferred_element_type=jnp.float32)
    d_ref[...] = jnp.dot(h, adv_ref[...], preferred_element_type=jnp.float32)


def _tc_proj1(x, w, asv, adv):
    n, d = x.shape
    c = w.shape[1]
    grid = n // _BLK
    return pl.pallas_call(
        _proj1_body,
        grid=(grid,),
        in_specs=[
            pl.BlockSpec((_BLK, d), lambda i: (i, 0)),
            pl.BlockSpec((d, c), lambda i: (0, 0)),
            pl.BlockSpec((c, 1), lambda i: (0, 0)),
            pl.BlockSpec((c, 1), lambda i: (0, 0)),
        ],
        out_specs=[
            pl.BlockSpec((_BLK, CP), lambda i: (i, 0)),
            pl.BlockSpec((_BLK, 1), lambda i: (i, 0)),
            pl.BlockSpec((_BLK, 1), lambda i: (i, 0)),
        ],
        out_shape=[
            jax.ShapeDtypeStruct((n, CP), jnp.float32),
            jax.ShapeDtypeStruct((n, 1), jnp.float32),
            jax.ShapeDtypeStruct((n, 1), jnp.float32),
        ],
    )(x, w, asv, adv)


def _gat_out(p_ref, b_ref, c):
    num = p_ref[0, :, :c] + p_ref[1, :, :c]
    den = p_ref[0, :, c:c + 1] + p_ref[1, :, c:c + 1]
    out = num / (den + 1e-16)
    return jax.nn.relu(out + b_ref[...])


def _proj2_body(p_ref, b_ref, w_ref, asv_ref, adv_ref, h_ref, s_ref, d_ref):
    c = w_ref.shape[0]
    hin = _gat_out(p_ref, b_ref, c)
    h = jnp.dot(hin, w_ref[...], preferred_element_type=jnp.float32)
    h_ref[...] = jnp.concatenate(
        [h, jnp.zeros((h.shape[0], CP - h.shape[1]), jnp.float32)], axis=1)
    s_ref[...] = jnp.dot(h, asv_ref[...], preferred_element_type=jnp.float32)
    d_ref[...] = jnp.dot(h, adv_ref[...], preferred_element_type=jnp.float32)


def _tc_proj2(p, b, w, asv, adv):
    _, n, _ = p.shape
    c = w.shape[0]
    c2 = w.shape[1]
    grid = n // _BLK
    return pl.pallas_call(
        _proj2_body,
        grid=(grid,),
        in_specs=[
            pl.BlockSpec((2, _BLK, CP), lambda i: (0, i, 0)),
            pl.BlockSpec((1, c), lambda i: (0, 0)),
            pl.BlockSpec((c, c2), lambda i: (0, 0)),
            pl.BlockSpec((c2, 1), lambda i: (0, 0)),
            pl.BlockSpec((c2, 1), lambda i: (0, 0)),
        ],
        out_specs=[
            pl.BlockSpec((_BLK, CP), lambda i: (i, 0)),
            pl.BlockSpec((_BLK, 1), lambda i: (i, 0)),
            pl.BlockSpec((_BLK, 1), lambda i: (i, 0)),
        ],
        out_shape=[
            jax.ShapeDtypeStruct((n, CP), jnp.float32),
            jax.ShapeDtypeStruct((n, 1), jnp.float32),
            jax.ShapeDtypeStruct((n, 1), jnp.float32),
        ],
    )(p, b, w, asv, adv)


def _final_body(p_ref, b_ref, wp_ref, bp_ref, bat_ref, out_ref, cnt_ref, acc_ref):
    i = pl.program_id(0)
    nsteps = pl.num_programs(0)
    g = cnt_ref.shape[0]
    c = wp_ref.shape[0]

    hin = _gat_out(p_ref, b_ref, c)
    sc = jnp.dot(hin, wp_ref[...], preferred_element_type=jnp.float32)
    sc = sc + bp_ref[...]
    iota_g = lax.broadcasted_iota(jnp.int32, (_BLK, g), 1)
    oh = (bat_ref[...] == iota_g).astype(jnp.float32)
    part = lax.dot_general(oh, sc, (((0,), (0,)), ((), ())),
                           preferred_element_type=jnp.float32)
    cnt = jnp.sum(oh, axis=0)[:, None]

    @pl.when(i == 0)
    def _():
        cnt_ref[...] = jnp.zeros_like(cnt_ref)
        acc_ref[...] = jnp.zeros_like(acc_ref)

    cnt_ref[...] += cnt
    acc_ref[...] += part

    @pl.when(i == nsteps - 1)
    def _():
        out_ref[...] = acc_ref[...] / jnp.maximum(cnt_ref[...], 1.0)


def _tc_final(p, b, wp, bp, bat, g):
    _, n, _ = p.shape
    c = wp.shape[0]
    out_c = wp.shape[1]
    grid = n // _BLK
    return pl.pallas_call(
        _final_body,
        grid=(grid,),
        in_specs=[
            pl.BlockSpec((2, _BLK, CP), lambda i: (0, i, 0)),
            pl.BlockSpec((1, c), lambda i: (0, 0)),
            pl.BlockSpec((c, out_c), lambda i: (0, 0)),
            pl.BlockSpec((1, out_c), lambda i: (0, 0)),
            pl.BlockSpec((_BLK, 1), lambda i: (i, 0)),
        ],
        out_specs=pl.BlockSpec((g, out_c), lambda i: (0, 0)),
        out_shape=jax.ShapeDtypeStruct((g, out_c), jnp.float32),
        scratch_shapes=[
            pltpu.VMEM((g, 1), jnp.float32),
            pltpu.VMEM((g, out_c), jnp.float32),
        ],
    )(p, b, wp, bp, bat)


@jax.jit
def kernel(x, edge_index, batch, W1, a_src1, a_dst1, b1,
           W2, a_src2, a_dst2, b2, Wp, bp):
    n, _ = x.shape
    e = edge_index.shape[1]
    c = W1.shape[1]
    g = 64

    nrows = e // K
    src2 = edge_index[0].reshape(NW, nrows // NW, K)
    dst2 = edge_index[1].reshape(NW, nrows // NW, K)

    sc_edge = _make_sc_edge(n, e, c)

    h1, as1, ad1 = _tc_proj1(x, W1, a_src1.reshape(-1, 1), a_dst1.reshape(-1, 1))
    p1 = sc_edge(src2, dst2, h1, as1.reshape(-1), ad1.reshape(-1))
    h2, as2, ad2 = _tc_proj2(p1, b1.reshape(1, -1), W2,
                             a_src2.reshape(-1, 1), a_dst2.reshape(-1, 1))
    p2 = sc_edge(src2, dst2, h2, as2.reshape(-1), ad2.reshape(-1))
    out = _tc_final(p2, b2.reshape(1, -1), Wp, bp.reshape(1, -1),
                    batch.reshape(-1, 1), g)
    return out


# speedup vs baseline: 2.1062x; 1.0847x over previous
import functools

import jax
import jax.numpy as jnp
from jax import lax
from jax.experimental import pallas as pl
from jax.experimental.pallas import tpu as pltpu
from jax.experimental.pallas import tpu_sc as plsc

NC = 2
NS = 16
NW = NC * NS
K = 80
L = 16
CP = 80
NB = 3
ZB = 20


def _make_sc_edge(n_nodes: int, n_edges: int, c_feat: int):
    assert n_edges % (NW * K) == 0
    nrows = n_edges // K
    r2 = nrows // NW
    zrow = 640
    n_full, n_rem = divmod(n_nodes, zrow)
    grp = K // L
    mesh = plsc.VectorSubcoreMesh(core_axis_name="c", subcore_axis_name="s",
                                  num_cores=NC, num_subcores=NS)

    @functools.partial(
        pl.kernel,
        out_type=jax.ShapeDtypeStruct((NC, n_nodes, 128), jnp.float32),
        mesh=mesh,
        scratch_types=dict(
            asrc_l=pltpu.VMEM((n_nodes,), jnp.float32),
            adst_l=pltpu.VMEM((n_nodes,), jnp.float32),
            src2_l=pltpu.VMEM((r2, K), jnp.int32),
            dst2_l=pltpu.VMEM((r2, K), jnp.int32),
            gbuf=pltpu.VMEM((NB, K, CP), jnp.float32),
            sbuf=pltpu.VMEM((NB, K, CP), jnp.float32),
            zbuf=pltpu.VMEM((ZB, CP), jnp.float32),
            out_sp=pltpu.VMEM_SHARED((n_nodes, CP), jnp.float32),
            gsem=pltpu.SemaphoreType.DMA((NB,)),
            ssem=pltpu.SemaphoreType.DMA((NB,)),
        ),
        compiler_params=pltpu.CompilerParams(needs_layout_passes=False,
                                            use_tc_tiling_on_sc=False),
    )
    def sc_edge(src2_hbm, dst2_hbm, h_hbm, asrc_hbm, adst_hbm, out_hbm,
                asrc_l, adst_l, src2_l, dst2_l, gbuf, sbuf, zbuf, out_sp,
                gsem, ssem):
        c = lax.axis_index("c")
        s = lax.axis_index("s")
        w = c * NS + s

        pltpu.sync_copy(asrc_hbm, asrc_l)
        pltpu.sync_copy(adst_hbm, adst_l)
        pltpu.sync_copy(src2_hbm.at[w], src2_l)
        pltpu.sync_copy(dst2_hbm.at[w], dst2_l)

        zero16 = jnp.zeros((L,), jnp.float32)
        for i in range(ZB):
            for rr in range(CP // L):
                zbuf[i, pl.ds(rr * L, L)] = zero16

        @pl.when(s < n_full)
        def _():
            def zc(i, cr):
                pltpu.sync_copy(zbuf, out_sp.at[pl.ds(s * zrow + i * ZB, ZB)])
                return cr
            lax.fori_loop(0, zrow // ZB, zc, 0)

        @pl.when(s == n_full)
        def _():
            if n_rem:
                def zc(i, cr):
                    pltpu.sync_copy(
                        zbuf, out_sp.at[pl.ds(n_full * zrow + i * ZB, ZB)])
                    return cr
                lax.fori_loop(0, n_rem // ZB, zc, 0)

        plsc.subcore_barrier()

        onehot0 = (lax.iota(jnp.int32, L) == 0).astype(jnp.float32)

        def scale_chunk(j, src, dst):
            for g in range(grp):
                sidx = src2_l[j, pl.ds(g * L, L)]
                didx = dst2_l[j, pl.ds(g * L, L)]
                e = (plsc.load_gather(asrc_l, [sidx])
                     + plsc.load_gather(adst_l, [didx]))
                e = jnp.maximum(e, 0.2 * e)
                exv = jnp.exp(e)
                for ii in range(L):
                    a = exv[ii]
                    i = g * L + ii
                    for rr in range(c_feat // L):
                        dst[i, pl.ds(rr * L, L)] = src[i, pl.ds(rr * L, L)] * a
                    dst[i, pl.ds(c_feat, L)] = onehot0 * a

        def issue_gather(j, b):
            return pltpu.async_copy(h_hbm.at[src2_l.at[j]], gbuf.at[b],
                                    gsem.at[b])

        def issue_scatter(j, b):
            return pltpu.async_copy(sbuf.at[b], out_sp.at[dst2_l.at[j]],
                                    ssem.at[b], add=True)

        def wait_gather(j, b):
            pltpu.make_async_copy(h_hbm.at[src2_l.at[j]], gbuf.at[b],
                                  gsem.at[b]).wait()

        def wait_scatter(j, b):
            pltpu.make_async_copy(sbuf.at[b], out_sp.at[dst2_l.at[j]],
                                  ssem.at[b]).wait()

        def prologue(b, carry):
            issue_gather(b, b)
            return carry

        lax.fori_loop(0, NB, prologue, 0)

        def main_body(j, carry):
            b = lax.rem(j, NB)
            pl.when(j >= NB)(lambda: wait_scatter(j, b))
            wait_gather(j, b)
            scale_chunk(j, gbuf.at[b], sbuf.at[b])
            def _prefetch():
                issue_gather(j + NB, b)
            pl.when(j + NB < r2)(_prefetch)
            issue_scatter(j, b)
            return carry

        lax.fori_loop(0, r2, main_body, 0)

        def epilogue(t, carry):
            j = r2 - NB + t
            wait_scatter(j, lax.rem(j, NB))
            return carry

        lax.fori_loop(0, NB, epilogue, 0)
        plsc.subcore_barrier()

        @pl.when(s < n_full)
        def _():
            pltpu.sync_copy(out_sp.at[pl.ds(s * zrow, zrow)],
                            out_hbm.at[c, pl.ds(s * zrow, zrow), pl.ds(0, CP)])

        @pl.when(s == n_full)
        def _():
            if n_rem:
                pltpu.sync_copy(
                    out_sp.at[pl.ds(n_full * zrow, n_rem)],
                    out_hbm.at[c, pl.ds(n_full * zrow, n_rem), pl.ds(0, CP)])

    return sc_edge


_BLK = 2000


def _proj1_body(x_ref, w_ref, asv_ref, adv_ref, h_ref, s_ref, d_ref):
    h = jnp.dot(x_ref[...], w_ref[...], preferred_element_type=jnp.float32)
    h_ref[...] = jnp.concatenate(
        [h, jnp.zeros((h.shape[0], CP - h.shape[1]), jnp.float32)], axis=1)
    s_ref[...] = jnp.dot(h, asv_ref[...], preferred_element_type=jnp.float32)
    d_ref[...] = jnp.dot(h, adv_ref[...], preferred_element_type=jnp.float32)


def _tc_proj1(x, w, asv, adv):
    n, d = x.shape
    c = w.shape[1]
    grid = n // _BLK
    return pl.pallas_call(
        _proj1_body,
        grid=(grid,),
        in_specs=[
            pl.BlockSpec((_BLK, d), lambda i: (i, 0)),
            pl.BlockSpec((d, c), lambda i: (0, 0)),
            pl.BlockSpec((c, 1), lambda i: (0, 0)),
            pl.BlockSpec((c, 1), lambda i: (0, 0)),
        ],
        out_specs=[
            pl.BlockSpec((_BLK, CP), lambda i: (i, 0)),
            pl.BlockSpec((_BLK, 1), lambda i: (i, 0)),
            pl.BlockSpec((_BLK, 1), lambda i: (i, 0)),
        ],
        out_shape=[
            jax.ShapeDtypeStruct((n, CP), jnp.float32),
            jax.ShapeDtypeStruct((n, 1), jnp.float32),
            jax.ShapeDtypeStruct((n, 1), jnp.float32),
        ],
    )(x, w, asv, adv)


def _gat_out(p_ref, b_ref, c):
    num = p_ref[0, :, :c] + p_ref[1, :, :c]
    den = p_ref[0, :, c:c + 1] + p_ref[1, :, c:c + 1]
    out = num / (den + 1e-16)
    return jax.nn.relu(out + b_ref[...])


def _proj2_body(p_ref, b_ref, w_ref, asv_ref, adv_ref, h_ref, s_ref, d_ref):
    c = w_ref.shape[0]
    hin = _gat_out(p_ref, b_ref, c)
    h = jnp.dot(hin, w_ref[...], preferred_element_type=jnp.float32)
    h_ref[...] = jnp.concatenate(
        [h, jnp.zeros((h.shape[0], CP - h.shape[1]), jnp.float32)], axis=1)
    s_ref[...] = jnp.dot(h, asv_ref[...], preferred_element_type=jnp.float32)
    d_ref[...] = jnp.dot(h, adv_ref[...], preferred_element_type=jnp.float32)


def _tc_proj2(p, b, w, asv, adv):
    _, n, _ = p.shape
    c = w.shape[0]
    c2 = w.shape[1]
    grid = n // _BLK
    return pl.pallas_call(
        _proj2_body,
        grid=(grid,),
        in_specs=[
            pl.BlockSpec((2, _BLK, 128), lambda i: (0, i, 0)),
            pl.BlockSpec((1, c), lambda i: (0, 0)),
            pl.BlockSpec((c, c2), lambda i: (0, 0)),
            pl.BlockSpec((c2, 1), lambda i: (0, 0)),
            pl.BlockSpec((c2, 1), lambda i: (0, 0)),
        ],
        out_specs=[
            pl.BlockSpec((_BLK, CP), lambda i: (i, 0)),
            pl.BlockSpec((_BLK, 1), lambda i: (i, 0)),
            pl.BlockSpec((_BLK, 1), lambda i: (i, 0)),
        ],
        out_shape=[
            jax.ShapeDtypeStruct((n, CP), jnp.float32),
            jax.ShapeDtypeStruct((n, 1), jnp.float32),
            jax.ShapeDtypeStruct((n, 1), jnp.float32),
        ],
    )(p, b, w, asv, adv)


def _final_body(p_ref, b_ref, wp_ref, bp_ref, bat_ref, out_ref, cnt_ref, acc_ref):
    i = pl.program_id(0)
    nsteps = pl.num_programs(0)
    g = cnt_ref.shape[0]
    c = wp_ref.shape[0]

    hin = _gat_out(p_ref, b_ref, c)
    sc = jnp.dot(hin, wp_ref[...], preferred_element_type=jnp.float32)
    sc = sc + bp_ref[...]
    iota_g = lax.broadcasted_iota(jnp.int32, (_BLK, g), 1)
    oh = (bat_ref[...] == iota_g).astype(jnp.float32)
    part = lax.dot_general(oh, sc, (((0,), (0,)), ((), ())),
                           preferred_element_type=jnp.float32)
    cnt = jnp.sum(oh, axis=0)[:, None]

    @pl.when(i == 0)
    def _():
        cnt_ref[...] = jnp.zeros_like(cnt_ref)
        acc_ref[...] = jnp.zeros_like(acc_ref)

    cnt_ref[...] += cnt
    acc_ref[...] += part

    @pl.when(i == nsteps - 1)
    def _():
        out_ref[...] = acc_ref[...] / jnp.maximum(cnt_ref[...], 1.0)


def _tc_final(p, b, wp, bp, bat, g):
    _, n, _ = p.shape
    c = wp.shape[0]
    out_c = wp.shape[1]
    grid = n // _BLK
    return pl.pallas_call(
        _final_body,
        grid=(grid,),
        in_specs=[
            pl.BlockSpec((2, _BLK, 128), lambda i: (0, i, 0)),
            pl.BlockSpec((1, c), lambda i: (0, 0)),
            pl.BlockSpec((c, out_c), lambda i: (0, 0)),
            pl.BlockSpec((1, out_c), lambda i: (0, 0)),
            pl.BlockSpec((_BLK, 1), lambda i: (i, 0)),
        ],
        out_specs=pl.BlockSpec((g, out_c), lambda i: (0, 0)),
        out_shape=jax.ShapeDtypeStruct((g, out_c), jnp.float32),
        scratch_shapes=[
            pltpu.VMEM((g, 1), jnp.float32),
            pltpu.VMEM((g, out_c), jnp.float32),
        ],
    )(p, b, wp, bp, bat)


@jax.jit
def kernel(x, edge_index, batch, W1, a_src1, a_dst1, b1,
           W2, a_src2, a_dst2, b2, Wp, bp):
    n, _ = x.shape
    e = edge_index.shape[1]
    c = W1.shape[1]
    g = 64

    nrows = e // K
    src2 = edge_index[0].reshape(NW, nrows // NW, K)
    dst2 = edge_index[1].reshape(NW, nrows // NW, K)

    sc_edge = _make_sc_edge(n, e, c)

    h1, as1, ad1 = _tc_proj1(x, W1, a_src1.reshape(-1, 1), a_dst1.reshape(-1, 1))
    p1 = sc_edge(src2, dst2, h1, as1.reshape(-1), ad1.reshape(-1))
    h2, as2, ad2 = _tc_proj2(p1, b1.reshape(1, -1), W2,
                             a_src2.reshape(-1, 1), a_dst2.reshape(-1, 1))
    p2 = sc_edge(src2, dst2, h2, as2.reshape(-1), ad2.reshape(-1))
    out = _tc_final(p2, b2.reshape(1, -1), Wp, bp.reshape(1, -1),
                    batch.reshape(-1, 1), g)
    return out


# speedup vs baseline: 2.2535x; 1.0700x over previous
import functools

import jax
import jax.numpy as jnp
from jax import lax
from jax.experimental import pallas as pl
from jax.experimental.pallas import tpu as pltpu
from jax.experimental.pallas import tpu_sc as plsc

NC = 2
NS = 16
NW = NC * NS
K = 80
L = 16
CP = 80
NB = 3
ZB = 20


def _make_sc_edge(n_nodes: int, n_edges: int, c_feat: int):
    assert n_edges % (NW * K) == 0
    nrows = n_edges // K
    r2 = nrows // NW
    zrow = 640
    n_full, n_rem = divmod(n_nodes, zrow)
    grp = K // L
    mesh = plsc.VectorSubcoreMesh(core_axis_name="c", subcore_axis_name="s",
                                  num_cores=NC, num_subcores=NS)

    @functools.partial(
        pl.kernel,
        out_type=jax.ShapeDtypeStruct((NC, n_nodes, 128), jnp.float32),
        mesh=mesh,
        scratch_types=dict(
            asrc_l=pltpu.VMEM((n_nodes,), jnp.float32),
            adst_l=pltpu.VMEM((n_nodes,), jnp.float32),
            src2_l=pltpu.VMEM((r2, K), jnp.int32),
            dst2_l=pltpu.VMEM((r2, K), jnp.int32),
            gbuf=pltpu.VMEM((NB, K, CP), jnp.float32),
            sbuf=pltpu.VMEM((NB, K, CP), jnp.float32),
            zbuf=pltpu.VMEM((ZB, CP), jnp.float32),
            out_sp=pltpu.VMEM_SHARED((n_nodes, CP), jnp.float32),
            gsem=pltpu.SemaphoreType.DMA((NB,)),
            ssem=pltpu.SemaphoreType.DMA((NB,)),
        ),
        compiler_params=pltpu.CompilerParams(needs_layout_passes=False,
                                            use_tc_tiling_on_sc=False),
    )
    def sc_edge(src2_hbm, dst2_hbm, h_hbm, asrc_hbm, adst_hbm, out_hbm,
                asrc_l, adst_l, src2_l, dst2_l, gbuf, sbuf, zbuf, out_sp,
                gsem, ssem):
        c = lax.axis_index("c")
        s = lax.axis_index("s")
        w = c * NS + s

        pltpu.sync_copy(asrc_hbm, asrc_l)
        pltpu.sync_copy(adst_hbm, adst_l)
        pltpu.sync_copy(src2_hbm.at[w], src2_l)
        pltpu.sync_copy(dst2_hbm.at[w], dst2_l)

        zero16 = jnp.zeros((L,), jnp.float32)
        for i in range(ZB):
            for rr in range(CP // L):
                zbuf[i, pl.ds(rr * L, L)] = zero16

        @pl.when(s < n_full)
        def _():
            def zc(i, cr):
                pltpu.sync_copy(zbuf, out_sp.at[pl.ds(s * zrow + i * ZB, ZB)])
                return cr
            lax.fori_loop(0, zrow // ZB, zc, 0)

        @pl.when(s == n_full)
        def _():
            if n_rem:
                def zc(i, cr):
                    pltpu.sync_copy(
                        zbuf, out_sp.at[pl.ds(n_full * zrow + i * ZB, ZB)])
                    return cr
                lax.fori_loop(0, n_rem // ZB, zc, 0)

        plsc.subcore_barrier()

        onehot0 = (lax.iota(jnp.int32, L) == 0).astype(jnp.float32)

        def scale_chunk(j, src, dst):
            for g in range(grp):
                sidx = src2_l[j, pl.ds(g * L, L)]
                didx = dst2_l[j, pl.ds(g * L, L)]
                e = (plsc.load_gather(asrc_l, [sidx])
                     + plsc.load_gather(adst_l, [didx]))
                e = jnp.maximum(e, 0.2 * e)
                exv = jnp.exp(e)
                for ii in range(L):
                    a = exv[ii]
                    i = g * L + ii
                    for rr in range(c_feat // L):
                        dst[i, pl.ds(rr * L, L)] = src[i, pl.ds(rr * L, L)] * a
                    dst[i, pl.ds(c_feat, L)] = onehot0 * a

        def issue_gather(j, b):
            return pltpu.async_copy(h_hbm.at[src2_l.at[j]], gbuf.at[b],
                                    gsem.at[b])

        def issue_scatter(j, b):
            return pltpu.async_copy(sbuf.at[b], out_sp.at[dst2_l.at[j]],
                                    ssem.at[b], add=True)

        def wait_gather(j, b):
            pltpu.make_async_copy(h_hbm.at[src2_l.at[j]], gbuf.at[b],
                                  gsem.at[b]).wait()

        def wait_scatter(j, b):
            pltpu.make_async_copy(sbuf.at[b], out_sp.at[dst2_l.at[j]],
                                  ssem.at[b]).wait()

        def prologue(b, carry):
            issue_gather(b, b)
            return carry

        lax.fori_loop(0, NB, prologue, 0)

        def main_body(j, carry):
            b = lax.rem(j, NB)
            pl.when(j >= NB)(lambda: wait_scatter(j, b))
            wait_gather(j, b)
            scale_chunk(j, gbuf.at[b], sbuf.at[b])
            def _prefetch():
                issue_gather(j + NB, b)
            pl.when(j + NB < r2)(_prefetch)
            issue_scatter(j, b)
            return carry

        lax.fori_loop(0, r2, main_body, 0)

        def epilogue(t, carry):
            j = r2 - NB + t
            wait_scatter(j, lax.rem(j, NB))
            return carry

        lax.fori_loop(0, NB, epilogue, 0)
        plsc.subcore_barrier()

        @pl.when(s < n_full)
        def _():
            pltpu.sync_copy(out_sp.at[pl.ds(s * zrow, zrow)],
                            out_hbm.at[c, pl.ds(s * zrow, zrow), pl.ds(0, CP)])

        @pl.when(s == n_full)
        def _():
            if n_rem:
                pltpu.sync_copy(
                    out_sp.at[pl.ds(n_full * zrow, n_rem)],
                    out_hbm.at[c, pl.ds(n_full * zrow, n_rem), pl.ds(0, CP)])

    return sc_edge


_BLK = 2000
_BLKP = 2048


def _proj1_body(x_ref, w_ref, asv_ref, adv_ref, h_ref, s_ref, d_ref):
    h = jnp.dot(x_ref[...], w_ref[...], preferred_element_type=jnp.float32)
    h_ref[...] = jnp.concatenate(
        [h, jnp.zeros((h.shape[0], CP - h.shape[1]), jnp.float32)], axis=1)
    s_ref[...] = jnp.sum(h * asv_ref[...], axis=1)
    d_ref[...] = jnp.sum(h * adv_ref[...], axis=1)


def _tc_proj1(x, w, asv, adv):
    n, d = x.shape
    c = w.shape[1]
    grid = (n + _BLKP - 1) // _BLKP
    return pl.pallas_call(
        _proj1_body,
        grid=(grid,),
        in_specs=[
            pl.BlockSpec((_BLKP, d), lambda i: (i, 0)),
            pl.BlockSpec((d, c), lambda i: (0, 0)),
            pl.BlockSpec((1, c), lambda i: (0, 0)),
            pl.BlockSpec((1, c), lambda i: (0, 0)),
        ],
        out_specs=[
            pl.BlockSpec((_BLKP, CP), lambda i: (i, 0)),
            pl.BlockSpec((_BLKP,), lambda i: (i,)),
            pl.BlockSpec((_BLKP,), lambda i: (i,)),
        ],
        out_shape=[
            jax.ShapeDtypeStruct((n, CP), jnp.float32),
            jax.ShapeDtypeStruct((n,), jnp.float32),
            jax.ShapeDtypeStruct((n,), jnp.float32),
        ],
    )(x, w, asv, adv)


def _gat_out(p_ref, b_ref, c):
    num = p_ref[0, :, :c] + p_ref[1, :, :c]
    den = p_ref[0, :, c:c + 1] + p_ref[1, :, c:c + 1]
    out = num / (den + 1e-16)
    return jax.nn.relu(out + b_ref[...])


def _proj2_body(p_ref, b_ref, w_ref, asv_ref, adv_ref, h_ref, s_ref, d_ref):
    c = w_ref.shape[0]
    hin = _gat_out(p_ref, b_ref, c)
    h = jnp.dot(hin, w_ref[...], preferred_element_type=jnp.float32)
    h_ref[...] = jnp.concatenate(
        [h, jnp.zeros((h.shape[0], CP - h.shape[1]), jnp.float32)], axis=1)
    s_ref[...] = jnp.sum(h * asv_ref[...], axis=1)
    d_ref[...] = jnp.sum(h * adv_ref[...], axis=1)


def _tc_proj2(p, b, w, asv, adv):
    _, n, _ = p.shape
    c = w.shape[0]
    c2 = w.shape[1]
    grid = (n + _BLKP - 1) // _BLKP
    return pl.pallas_call(
        _proj2_body,
        grid=(grid,),
        in_specs=[
            pl.BlockSpec((2, _BLKP, 128), lambda i: (0, i, 0)),
            pl.BlockSpec((1, c), lambda i: (0, 0)),
            pl.BlockSpec((c, c2), lambda i: (0, 0)),
            pl.BlockSpec((1, c2), lambda i: (0, 0)),
            pl.BlockSpec((1, c2), lambda i: (0, 0)),
        ],
        out_specs=[
            pl.BlockSpec((_BLKP, CP), lambda i: (i, 0)),
            pl.BlockSpec((_BLKP,), lambda i: (i,)),
            pl.BlockSpec((_BLKP,), lambda i: (i,)),
        ],
        out_shape=[
            jax.ShapeDtypeStruct((n, CP), jnp.float32),
            jax.ShapeDtypeStruct((n,), jnp.float32),
            jax.ShapeDtypeStruct((n,), jnp.float32),
        ],
    )(p, b, w, asv, adv)


def _final_body(p_ref, b_ref, wp_ref, bp_ref, bat_ref, out_ref, cnt_ref, acc_ref):
    i = pl.program_id(0)
    nsteps = pl.num_programs(0)
    g = cnt_ref.shape[0]
    c = wp_ref.shape[0]

    hin = _gat_out(p_ref, b_ref, c)
    sc = jnp.dot(hin, wp_ref[...], preferred_element_type=jnp.float32)
    sc = sc + bp_ref[...]
    iota_g = lax.broadcasted_iota(jnp.int32, (_BLK, g), 1)
    oh = (bat_ref[...] == iota_g).astype(jnp.float32)
    part = lax.dot_general(oh, sc, (((0,), (0,)), ((), ())),
                           preferred_element_type=jnp.float32)
    cnt = jnp.sum(oh, axis=0)[:, None]

    @pl.when(i == 0)
    def _():
        cnt_ref[...] = jnp.zeros_like(cnt_ref)
        acc_ref[...] = jnp.zeros_like(acc_ref)

    cnt_ref[...] += cnt
    acc_ref[...] += part

    @pl.when(i == nsteps - 1)
    def _():
        out_ref[...] = acc_ref[...] / jnp.maximum(cnt_ref[...], 1.0)


def _tc_final(p, b, wp, bp, bat, g):
    _, n, _ = p.shape
    c = wp.shape[0]
    out_c = wp.shape[1]
    grid = n // _BLK
    return pl.pallas_call(
        _final_body,
        grid=(grid,),
        in_specs=[
            pl.BlockSpec((2, _BLK, 128), lambda i: (0, i, 0)),
            pl.BlockSpec((1, c), lambda i: (0, 0)),
            pl.BlockSpec((c, out_c), lambda i: (0, 0)),
            pl.BlockSpec((1, out_c), lambda i: (0, 0)),
            pl.BlockSpec((_BLK, 1), lambda i: (i, 0)),
        ],
        out_specs=pl.BlockSpec((g, out_c), lambda i: (0, 0)),
        out_shape=jax.ShapeDtypeStruct((g, out_c), jnp.float32),
        scratch_shapes=[
            pltpu.VMEM((g, 1), jnp.float32),
            pltpu.VMEM((g, out_c), jnp.float32),
        ],
    )(p, b, wp, bp, bat)


@jax.jit
def kernel(x, edge_index, batch, W1, a_src1, a_dst1, b1,
           W2, a_src2, a_dst2, b2, Wp, bp):
    n, _ = x.shape
    e = edge_index.shape[1]
    c = W1.shape[1]
    g = 64

    nrows = e // K
    src2 = edge_index[0].reshape(NW, nrows // NW, K)
    dst2 = edge_index[1].reshape(NW, nrows // NW, K)

    sc_edge = _make_sc_edge(n, e, c)

    h1, as1, ad1 = _tc_proj1(x, W1, a_src1.reshape(1, -1), a_dst1.reshape(1, -1))
    p1 = sc_edge(src2, dst2, h1, as1, ad1)
    h2, as2, ad2 = _tc_proj2(p1, b1.reshape(1, -1), W2,
                             a_src2.reshape(1, -1), a_dst2.reshape(1, -1))
    p2 = sc_edge(src2, dst2, h2, as2, ad2)
    out = _tc_final(p2, b2.reshape(1, -1), Wp, bp.reshape(1, -1),
                    batch.reshape(-1, 1), g)
    return out


# speedup vs baseline: 2.3767x; 1.0547x over previous
import functools

import jax
import jax.numpy as jnp
from jax import lax
from jax.experimental import pallas as pl
from jax.experimental.pallas import tpu as pltpu
from jax.experimental.pallas import tpu_sc as plsc

NC = 2
NS = 16
NW = NC * NS
K = 80
L = 16
CP = 80
NB = 3
ZB = 20


def _make_sc_edge(n_nodes: int, n_edges: int, c_feat: int):
    assert n_edges % (NW * K) == 0
    nrows = n_edges // K
    r2 = nrows // NW
    zrow = 640
    n_full, n_rem = divmod(n_nodes, zrow)
    grp = K // L
    mesh = plsc.VectorSubcoreMesh(core_axis_name="c", subcore_axis_name="s",
                                  num_cores=NC, num_subcores=NS)

    @functools.partial(
        pl.kernel,
        out_type=jax.ShapeDtypeStruct((NC, n_nodes, 128), jnp.float32),
        mesh=mesh,
        scratch_types=dict(
            asrc_l=pltpu.VMEM((n_nodes,), jnp.float32),
            adst_l=pltpu.VMEM((n_nodes,), jnp.float32),
            src2_l=pltpu.VMEM((r2, K), jnp.int32),
            dst2_l=pltpu.VMEM((r2, K), jnp.int32),
            gbuf=pltpu.VMEM((NB, K, CP), jnp.float32),
            sbuf=pltpu.VMEM((NB, K, CP), jnp.float32),
            zbuf=pltpu.VMEM((ZB, CP), jnp.float32),
            out_sp=pltpu.VMEM_SHARED((n_nodes, CP), jnp.float32),
            gsem=pltpu.SemaphoreType.DMA((NB,)),
            ssem=pltpu.SemaphoreType.DMA((NB,)),
            zsem=pltpu.SemaphoreType.DMA,
        ),
        compiler_params=pltpu.CompilerParams(needs_layout_passes=False,
                                            use_tc_tiling_on_sc=False),
    )
    def sc_edge(src2_hbm, dst2_hbm, h_hbm, asrc_hbm, adst_hbm, out_hbm,
                asrc_l, adst_l, src2_l, dst2_l, gbuf, sbuf, zbuf, out_sp,
                gsem, ssem, zsem):
        c = lax.axis_index("c")
        s = lax.axis_index("s")
        w = c * NS + s

        d_as = pltpu.async_copy(asrc_hbm, asrc_l, gsem.at[0])
        d_ad = pltpu.async_copy(adst_hbm, adst_l, gsem.at[1])
        d_s2 = pltpu.async_copy(src2_hbm.at[w], src2_l, ssem.at[0])
        d_d2 = pltpu.async_copy(dst2_hbm.at[w], dst2_l, ssem.at[1])

        zero16 = jnp.zeros((L,), jnp.float32)
        for i in range(ZB):
            for rr in range(CP // L):
                zbuf[i, pl.ds(rr * L, L)] = zero16

        nz = lax.select(s < n_full, zrow // ZB,
                        lax.select(s == n_full, n_rem // ZB, 0))
        zbase = lax.min(s, n_full) * zrow

        def zc(i, cr):
            pltpu.async_copy(zbuf, out_sp.at[pl.ds(zbase + i * ZB, ZB)], zsem)
            return cr
        lax.fori_loop(0, nz, zc, 0)

        def zw(i, cr):
            pltpu.make_async_copy(
                zbuf, out_sp.at[pl.ds(zbase + i * ZB, ZB)], zsem).wait()
            return cr
        lax.fori_loop(0, nz, zw, 0)

        d_as.wait()
        d_ad.wait()
        d_s2.wait()
        d_d2.wait()
        plsc.subcore_barrier()

        onehot0 = (lax.iota(jnp.int32, L) == 0).astype(jnp.float32)

        def scale_chunk(j, src, dst):
            for g in range(grp):
                sidx = src2_l[j, pl.ds(g * L, L)]
                didx = dst2_l[j, pl.ds(g * L, L)]
                e = (plsc.load_gather(asrc_l, [sidx])
                     + plsc.load_gather(adst_l, [didx]))
                e = jnp.maximum(e, 0.2 * e)
                exv = jnp.exp(e)
                for ii in range(L):
                    a = exv[ii]
                    i = g * L + ii
                    for rr in range(c_feat // L):
                        dst[i, pl.ds(rr * L, L)] = src[i, pl.ds(rr * L, L)] * a
                    dst[i, pl.ds(c_feat, L)] = onehot0 * a

        def issue_gather(j, b):
            return pltpu.async_copy(h_hbm.at[src2_l.at[j]], gbuf.at[b],
                                    gsem.at[b])

        def issue_scatter(j, b):
            return pltpu.async_copy(sbuf.at[b], out_sp.at[dst2_l.at[j]],
                                    ssem.at[b], add=True)

        def wait_gather(j, b):
            pltpu.make_async_copy(h_hbm.at[src2_l.at[j]], gbuf.at[b],
                                  gsem.at[b]).wait()

        def wait_scatter(j, b):
            pltpu.make_async_copy(sbuf.at[b], out_sp.at[dst2_l.at[j]],
                                  ssem.at[b]).wait()

        def prologue(b, carry):
            issue_gather(b, b)
            return carry

        lax.fori_loop(0, NB, prologue, 0)

        def main_body(j, carry):
            b = lax.rem(j, NB)
            pl.when(j >= NB)(lambda: wait_scatter(j, b))
            wait_gather(j, b)
            scale_chunk(j, gbuf.at[b], sbuf.at[b])
            def _prefetch():
                issue_gather(j + NB, b)
            pl.when(j + NB < r2)(_prefetch)
            issue_scatter(j, b)
            return carry

        lax.fori_loop(0, r2, main_body, 0)

        def epilogue(t, carry):
            j = r2 - NB + t
            wait_scatter(j, lax.rem(j, NB))
            return carry

        lax.fori_loop(0, NB, epilogue, 0)
        plsc.subcore_barrier()

        @pl.when(s < n_full)
        def _():
            pltpu.sync_copy(out_sp.at[pl.ds(s * zrow, zrow)],
                            out_hbm.at[c, pl.ds(s * zrow, zrow), pl.ds(0, CP)])

        @pl.when(s == n_full)
        def _():
            if n_rem:
                pltpu.sync_copy(
                    out_sp.at[pl.ds(n_full * zrow, n_rem)],
                    out_hbm.at[c, pl.ds(n_full * zrow, n_rem), pl.ds(0, CP)])

    return sc_edge


_BLK = 2000
_BLKP = 2048


def _proj1_body(x_ref, w_ref, asv_ref, adv_ref, h_ref, s_ref, d_ref):
    h = jnp.dot(x_ref[...], w_ref[...], preferred_element_type=jnp.float32)
    h_ref[...] = jnp.concatenate(
        [h, jnp.zeros((h.shape[0], CP - h.shape[1]), jnp.float32)], axis=1)
    s_ref[...] = jnp.sum(h * asv_ref[...], axis=1)
    d_ref[...] = jnp.sum(h * adv_ref[...], axis=1)


def _tc_proj1(x, w, asv, adv):
    n, d = x.shape
    c = w.shape[1]
    grid = (n + _BLKP - 1) // _BLKP
    return pl.pallas_call(
        _proj1_body,
        grid=(grid,),
        in_specs=[
            pl.BlockSpec((_BLKP, d), lambda i: (i, 0)),
            pl.BlockSpec((d, c), lambda i: (0, 0)),
            pl.BlockSpec((1, c), lambda i: (0, 0)),
            pl.BlockSpec((1, c), lambda i: (0, 0)),
        ],
        out_specs=[
            pl.BlockSpec((_BLKP, CP), lambda i: (i, 0)),
            pl.BlockSpec((_BLKP,), lambda i: (i,)),
            pl.BlockSpec((_BLKP,), lambda i: (i,)),
        ],
        out_shape=[
            jax.ShapeDtypeStruct((n, CP), jnp.float32),
            jax.ShapeDtypeStruct((n,), jnp.float32),
            jax.ShapeDtypeStruct((n,), jnp.float32),
        ],
    )(x, w, asv, adv)


def _gat_out(p_ref, b_ref, c):
    num = p_ref[0, :, :c] + p_ref[1, :, :c]
    den = p_ref[0, :, c:c + 1] + p_ref[1, :, c:c + 1]
    out = num / (den + 1e-16)
    return jax.nn.relu(out + b_ref[...])


def _proj2_body(p_ref, b_ref, w_ref, asv_ref, adv_ref, h_ref, s_ref, d_ref):
    c = w_ref.shape[0]
    hin = _gat_out(p_ref, b_ref, c)
    h = jnp.dot(hin, w_ref[...], preferred_element_type=jnp.float32)
    h_ref[...] = jnp.concatenate(
        [h, jnp.zeros((h.shape[0], CP - h.shape[1]), jnp.float32)], axis=1)
    s_ref[...] = jnp.sum(h * asv_ref[...], axis=1)
    d_ref[...] = jnp.sum(h * adv_ref[...], axis=1)


def _tc_proj2(p, b, w, asv, adv):
    _, n, _ = p.shape
    c = w.shape[0]
    c2 = w.shape[1]
    grid = (n + _BLKP - 1) // _BLKP
    return pl.pallas_call(
        _proj2_body,
        grid=(grid,),
        in_specs=[
            pl.BlockSpec((2, _BLKP, 128), lambda i: (0, i, 0)),
            pl.BlockSpec((1, c), lambda i: (0, 0)),
            pl.BlockSpec((c, c2), lambda i: (0, 0)),
            pl.BlockSpec((1, c2), lambda i: (0, 0)),
            pl.BlockSpec((1, c2), lambda i: (0, 0)),
        ],
        out_specs=[
            pl.BlockSpec((_BLKP, CP), lambda i: (i, 0)),
            pl.BlockSpec((_BLKP,), lambda i: (i,)),
            pl.BlockSpec((_BLKP,), lambda i: (i,)),
        ],
        out_shape=[
            jax.ShapeDtypeStruct((n, CP), jnp.float32),
            jax.ShapeDtypeStruct((n,), jnp.float32),
            jax.ShapeDtypeStruct((n,), jnp.float32),
        ],
    )(p, b, w, asv, adv)


def _final_body(p_ref, b_ref, wp_ref, bp_ref, bat_ref, out_ref, cnt_ref, acc_ref):
    i = pl.program_id(0)
    nsteps = pl.num_programs(0)
    g = cnt_ref.shape[0]
    c = wp_ref.shape[0]

    hin = _gat_out(p_ref, b_ref, c)
    sc = jnp.dot(hin, wp_ref[...], preferred_element_type=jnp.float32)
    sc = sc + bp_ref[...]
    iota_g = lax.broadcasted_iota(jnp.int32, (_BLK, g), 1)
    oh = (bat_ref[...] == iota_g).astype(jnp.float32)
    part = lax.dot_general(oh, sc, (((0,), (0,)), ((), ())),
                           preferred_element_type=jnp.float32)
    cnt = jnp.sum(oh, axis=0)[:, None]

    @pl.when(i == 0)
    def _():
        cnt_ref[...] = jnp.zeros_like(cnt_ref)
        acc_ref[...] = jnp.zeros_like(acc_ref)

    cnt_ref[...] += cnt
    acc_ref[...] += part

    @pl.when(i == nsteps - 1)
    def _():
        out_ref[...] = acc_ref[...] / jnp.maximum(cnt_ref[...], 1.0)


def _tc_final(p, b, wp, bp, bat, g):
    _, n, _ = p.shape
    c = wp.shape[0]
    out_c = wp.shape[1]
    grid = n // _BLK
    return pl.pallas_call(
        _final_body,
        grid=(grid,),
        in_specs=[
            pl.BlockSpec((2, _BLK, 128), lambda i: (0, i, 0)),
            pl.BlockSpec((1, c), lambda i: (0, 0)),
            pl.BlockSpec((c, out_c), lambda i: (0, 0)),
            pl.BlockSpec((1, out_c), lambda i: (0, 0)),
            pl.BlockSpec((_BLK, 1), lambda i: (i, 0)),
        ],
        out_specs=pl.BlockSpec((g, out_c), lambda i: (0, 0)),
        out_shape=jax.ShapeDtypeStruct((g, out_c), jnp.float32),
        scratch_shapes=[
            pltpu.VMEM((g, 1), jnp.float32),
            pltpu.VMEM((g, out_c), jnp.float32),
        ],
    )(p, b, wp, bp, bat)


@jax.jit
def kernel(x, edge_index, batch, W1, a_src1, a_dst1, b1,
           W2, a_src2, a_dst2, b2, Wp, bp):
    n, _ = x.shape
    e = edge_index.shape[1]
    c = W1.shape[1]
    g = 64

    nrows = e // K
    src2 = edge_index[0].reshape(NW, nrows // NW, K)
    dst2 = edge_index[1].reshape(NW, nrows // NW, K)

    sc_edge = _make_sc_edge(n, e, c)

    h1, as1, ad1 = _tc_proj1(x, W1, a_src1.reshape(1, -1), a_dst1.reshape(1, -1))
    p1 = sc_edge(src2, dst2, h1, as1, ad1)
    h2, as2, ad2 = _tc_proj2(p1, b1.reshape(1, -1), W2,
                             a_src2.reshape(1, -1), a_dst2.reshape(1, -1))
    p2 = sc_edge(src2, dst2, h2, as2, ad2)
    out = _tc_final(p2, b2.reshape(1, -1), Wp, bp.reshape(1, -1),
                    batch.reshape(-1, 1), g)
    return out


# speedup vs baseline: 2.3996x; 1.0096x over previous
import functools

import jax
import jax.numpy as jnp
from jax import lax
from jax.experimental import pallas as pl
from jax.experimental.pallas import tpu as pltpu
from jax.experimental.pallas import tpu_sc as plsc

NC = 2
NS = 16
NW = NC * NS
K = 80
L = 16
CP = 80
NB = 3
ZB = 20


def _make_sc_edge(n_nodes: int, n_edges: int, c_feat: int):
    assert n_edges % (NW * K) == 0
    nrows = n_edges // K
    r2 = nrows // NW
    zrow = 640
    n_full, n_rem = divmod(n_nodes, zrow)
    grp = K // L
    mesh = plsc.VectorSubcoreMesh(core_axis_name="c", subcore_axis_name="s",
                                  num_cores=NC, num_subcores=NS)

    @functools.partial(
        pl.kernel,
        out_type=jax.ShapeDtypeStruct((NC, n_nodes, 128), jnp.float32),
        mesh=mesh,
        scratch_types=dict(
            asrc_l=pltpu.VMEM((n_nodes,), jnp.float32),
            adst_l=pltpu.VMEM((n_nodes,), jnp.float32),
            src2_l=pltpu.VMEM((r2, K), jnp.int32),
            dst2_l=pltpu.VMEM((r2, K), jnp.int32),
            gbuf=pltpu.VMEM((NB, K, CP), jnp.float32),
            sbuf=pltpu.VMEM((NB, K, CP), jnp.float32),
            zbuf=pltpu.VMEM((ZB, CP), jnp.float32),
            out_sp=pltpu.VMEM_SHARED((n_nodes, CP), jnp.float32),
            gsem=pltpu.SemaphoreType.DMA((NB,)),
            ssem=pltpu.SemaphoreType.DMA((NB,)),
            zsem=pltpu.SemaphoreType.DMA,
        ),
        compiler_params=pltpu.CompilerParams(needs_layout_passes=False,
                                            use_tc_tiling_on_sc=False),
    )
    def sc_edge(src2_hbm, dst2_hbm, h_hbm, asrc_hbm, adst_hbm, out_hbm,
                asrc_l, adst_l, src2_l, dst2_l, gbuf, sbuf, zbuf, out_sp,
                gsem, ssem, zsem):
        c = lax.axis_index("c")
        s = lax.axis_index("s")
        w = c * NS + s

        d_as = pltpu.async_copy(asrc_hbm, asrc_l, gsem.at[0])
        d_ad = pltpu.async_copy(adst_hbm, adst_l, gsem.at[1])
        d_s2 = pltpu.async_copy(src2_hbm.at[w], src2_l, ssem.at[0])
        d_d2 = pltpu.async_copy(dst2_hbm.at[w], dst2_l, ssem.at[1])

        zero16 = jnp.zeros((L,), jnp.float32)
        for i in range(ZB):
            for rr in range(CP // L):
                zbuf[i, pl.ds(rr * L, L)] = zero16

        nz = lax.select(s < n_full, zrow // ZB,
                        lax.select(s == n_full, n_rem // ZB, 0))
        zbase = lax.min(s, n_full) * zrow

        def zc(i, cr):
            pltpu.async_copy(zbuf, out_sp.at[pl.ds(zbase + i * ZB, ZB)], zsem)
            return cr
        lax.fori_loop(0, nz, zc, 0)

        def zw(i, cr):
            pltpu.make_async_copy(
                zbuf, out_sp.at[pl.ds(zbase + i * ZB, ZB)], zsem).wait()
            return cr
        lax.fori_loop(0, nz, zw, 0)

        d_as.wait()
        d_ad.wait()
        d_s2.wait()
        d_d2.wait()
        plsc.subcore_barrier()

        onehot0 = (lax.iota(jnp.int32, L) == 0).astype(jnp.float32)

        def scale_chunk(j, src, dst):
            for g in range(grp):
                sidx = src2_l[j, pl.ds(g * L, L)]
                didx = dst2_l[j, pl.ds(g * L, L)]
                e = (plsc.load_gather(asrc_l, [sidx])
                     + plsc.load_gather(adst_l, [didx]))
                e = jnp.maximum(e, 0.2 * e)
                exv = jnp.exp(e)
                for ii in range(L):
                    a = exv[ii]
                    i = g * L + ii
                    for rr in range(c_feat // L):
                        dst[i, pl.ds(rr * L, L)] = src[i, pl.ds(rr * L, L)] * a
                    dst[i, pl.ds(c_feat, L)] = onehot0 * a

        def issue_gather(j, b):
            return pltpu.async_copy(h_hbm.at[src2_l.at[j]], gbuf.at[b],
                                    gsem.at[b])

        def issue_scatter(j, b):
            return pltpu.async_copy(sbuf.at[b], out_sp.at[dst2_l.at[j]],
                                    ssem.at[b], add=True)

        def wait_gather(j, b):
            pltpu.make_async_copy(h_hbm.at[src2_l.at[j]], gbuf.at[b],
                                  gsem.at[b]).wait()

        def wait_scatter(j, b):
            pltpu.make_async_copy(sbuf.at[b], out_sp.at[dst2_l.at[j]],
                                  ssem.at[b]).wait()

        def prologue(b, carry):
            issue_gather(b, b)
            return carry

        lax.fori_loop(0, NB, prologue, 0)

        def main_body(j, carry):
            b = lax.rem(j, NB)
            pl.when(j >= NB)(lambda: wait_scatter(j, b))
            wait_gather(j, b)
            scale_chunk(j, gbuf.at[b], sbuf.at[b])
            def _prefetch():
                issue_gather(j + NB, b)
            pl.when(j + NB < r2)(_prefetch)
            issue_scatter(j, b)
            return carry

        lax.fori_loop(0, r2, main_body, 0)

        def epilogue(t, carry):
            j = r2 - NB + t
            wait_scatter(j, lax.rem(j, NB))
            return carry

        lax.fori_loop(0, NB, epilogue, 0)
        plsc.subcore_barrier()

        @pl.when(s < n_full)
        def _():
            pltpu.sync_copy(out_sp.at[pl.ds(s * zrow, zrow)],
                            out_hbm.at[c, pl.ds(s * zrow, zrow), pl.ds(0, CP)])

        @pl.when(s == n_full)
        def _():
            if n_rem:
                pltpu.sync_copy(
                    out_sp.at[pl.ds(n_full * zrow, n_rem)],
                    out_hbm.at[c, pl.ds(n_full * zrow, n_rem), pl.ds(0, CP)])

    return sc_edge


_BLK = 2000
_BLKP = 2048


def _proj1_body(x_ref, w_ref, asv_ref, adv_ref, h_ref, s_ref, d_ref):
    h = jnp.dot(x_ref[...], w_ref[...], preferred_element_type=jnp.float32)
    h_ref[...] = jnp.concatenate(
        [h, jnp.zeros((h.shape[0], CP - h.shape[1]), jnp.float32)], axis=1)
    s_ref[...] = jnp.sum(h * asv_ref[...], axis=1)
    d_ref[...] = jnp.sum(h * adv_ref[...], axis=1)


def _tc_proj1(x, w, asv, adv):
    n, d = x.shape
    c = w.shape[1]
    grid = (n + _BLKP - 1) // _BLKP
    return pl.pallas_call(
        _proj1_body,
        grid=(grid,),
        in_specs=[
            pl.BlockSpec((_BLKP, d), lambda i: (i, 0)),
            pl.BlockSpec((d, c), lambda i: (0, 0)),
            pl.BlockSpec((1, c), lambda i: (0, 0)),
            pl.BlockSpec((1, c), lambda i: (0, 0)),
        ],
        out_specs=[
            pl.BlockSpec((_BLKP, CP), lambda i: (i, 0)),
            pl.BlockSpec((_BLKP,), lambda i: (i,)),
            pl.BlockSpec((_BLKP,), lambda i: (i,)),
        ],
        out_shape=[
            jax.ShapeDtypeStruct((n, CP), jnp.float32),
            jax.ShapeDtypeStruct((n,), jnp.float32),
            jax.ShapeDtypeStruct((n,), jnp.float32),
        ],
    )(x, w, asv, adv)


def _gat_out(p_ref, b_ref, c):
    num = p_ref[0, :, :c] + p_ref[1, :, :c]
    den = p_ref[0, :, c:c + 1] + p_ref[1, :, c:c + 1]
    out = num / (den + 1e-16)
    return jax.nn.relu(out + b_ref[...])


def _proj2_body(p_ref, b_ref, w_ref, asv_ref, adv_ref, h_ref, s_ref, d_ref):
    c = w_ref.shape[0]
    hin = _gat_out(p_ref, b_ref, c)
    h = jnp.dot(hin, w_ref[...], preferred_element_type=jnp.float32)
    h_ref[...] = jnp.concatenate(
        [h, jnp.zeros((h.shape[0], CP - h.shape[1]), jnp.float32)], axis=1)
    s_ref[...] = jnp.sum(h * asv_ref[...], axis=1)
    d_ref[...] = jnp.sum(h * adv_ref[...], axis=1)


def _tc_proj2(p, b, w, asv, adv):
    _, n, _ = p.shape
    c = w.shape[0]
    c2 = w.shape[1]
    grid = (n + _BLKP - 1) // _BLKP
    return pl.pallas_call(
        _proj2_body,
        grid=(grid,),
        in_specs=[
            pl.BlockSpec((2, _BLKP, 128), lambda i: (0, i, 0)),
            pl.BlockSpec((1, c), lambda i: (0, 0)),
            pl.BlockSpec((c, c2), lambda i: (0, 0)),
            pl.BlockSpec((1, c2), lambda i: (0, 0)),
            pl.BlockSpec((1, c2), lambda i: (0, 0)),
        ],
        out_specs=[
            pl.BlockSpec((_BLKP, CP), lambda i: (i, 0)),
            pl.BlockSpec((_BLKP,), lambda i: (i,)),
            pl.BlockSpec((_BLKP,), lambda i: (i,)),
        ],
        out_shape=[
            jax.ShapeDtypeStruct((n, CP), jnp.float32),
            jax.ShapeDtypeStruct((n,), jnp.float32),
            jax.ShapeDtypeStruct((n,), jnp.float32),
        ],
    )(p, b, w, asv, adv)


def _final_body(p_ref, b_ref, wp_ref, bp_ref, bat_ref, n_ref, out_ref,
                cnt_ref, acc_ref):
    i = pl.program_id(0)
    nsteps = pl.num_programs(0)
    g = cnt_ref.shape[0]
    c = wp_ref.shape[0]
    n = n_ref[0]

    row = i * _BLKP + lax.broadcasted_iota(jnp.int32, (_BLKP, 1), 0)
    vmask = row < n
    valid = vmask.astype(jnp.float32)
    hin = jnp.where(vmask, _gat_out(p_ref, b_ref, c), 0.0)
    sc = jnp.dot(hin, wp_ref[...], preferred_element_type=jnp.float32)
    sc = sc + bp_ref[...]
    iota_g = lax.broadcasted_iota(jnp.int32, (_BLKP, g), 1)
    oh = (bat_ref[...][:, None] == iota_g).astype(jnp.float32) * valid
    part = lax.dot_general(oh, sc, (((0,), (0,)), ((), ())),
                           preferred_element_type=jnp.float32)
    cnt = jnp.sum(oh, axis=0)[:, None]

    @pl.when(i == 0)
    def _():
        cnt_ref[...] = jnp.zeros_like(cnt_ref)
        acc_ref[...] = jnp.zeros_like(acc_ref)

    cnt_ref[...] += cnt
    acc_ref[...] += part

    @pl.when(i == nsteps - 1)
    def _():
        out_ref[...] = acc_ref[...] / jnp.maximum(cnt_ref[...], 1.0)


def _tc_final(p, b, wp, bp, bat, g):
    _, n, _ = p.shape
    c = wp.shape[0]
    out_c = wp.shape[1]
    grid = (n + _BLKP - 1) // _BLKP
    nn = jnp.full((1,), n, jnp.int32)
    return pl.pallas_call(
        _final_body,
        grid=(grid,),
        in_specs=[
            pl.BlockSpec((2, _BLKP, 128), lambda i: (0, i, 0)),
            pl.BlockSpec((1, c), lambda i: (0, 0)),
            pl.BlockSpec((c, out_c), lambda i: (0, 0)),
            pl.BlockSpec((1, out_c), lambda i: (0, 0)),
            pl.BlockSpec((_BLKP,), lambda i: (i,)),
            pl.BlockSpec(memory_space=pltpu.SMEM),
        ],
        out_specs=pl.BlockSpec((g, out_c), lambda i: (0, 0)),
        out_shape=jax.ShapeDtypeStruct((g, out_c), jnp.float32),
        scratch_shapes=[
            pltpu.VMEM((g, 1), jnp.float32),
            pltpu.VMEM((g, out_c), jnp.float32),
        ],
    )(p, b, wp, bp, bat, nn)


@jax.jit
def kernel(x, edge_index, batch, W1, a_src1, a_dst1, b1,
           W2, a_src2, a_dst2, b2, Wp, bp):
    n, _ = x.shape
    e = edge_index.shape[1]
    c = W1.shape[1]
    g = 64

    nrows = e // K
    src2 = edge_index[0].reshape(NW, nrows // NW, K)
    dst2 = edge_index[1].reshape(NW, nrows // NW, K)

    sc_edge = _make_sc_edge(n, e, c)

    h1, as1, ad1 = _tc_proj1(x, W1, a_src1.reshape(1, -1), a_dst1.reshape(1, -1))
    p1 = sc_edge(src2, dst2, h1, as1, ad1)
    h2, as2, ad2 = _tc_proj2(p1, b1.reshape(1, -1), W2,
                             a_src2.reshape(1, -1), a_dst2.reshape(1, -1))
    p2 = sc_edge(src2, dst2, h2, as2, ad2)
    out = _tc_final(p2, b2.reshape(1, -1), Wp, bp.reshape(1, -1), batch, g)
    return out
